# baseline (speedup 1.0000x reference)
import functools

import jax
import jax.numpy as jnp
import numpy as np
from jax import lax
from jax.experimental import pallas as pl
from jax.experimental.pallas import tpu as pltpu
from jax.experimental.pallas import tpu_sc as plsc

_NUM_CLASSES = 80
_STRIDES = (32.0, 16.0, 8.0)
_SIZES = (13, 26, 52)
_NA = 3
_BS = 4
_SCORE_THRESH = 0.01
_NMS_THRESH = 0.45
_PRE = 1024
_PRE_REAL = 1000
_TOPK = 200
_NV = sum(_NA * s * s for s in _SIZES)
_NP = 10752
_CAP = 2048
_LOCAL = 2048
_REG = _CAP + 2048
_BISECT_ITERS = 14


def _decode_body(p_ref, aw_ref, ah_ref, gx_ref, gy_ref, sv_ref, valid_ref,
                 boxes_ref, s_ref, t_ref):
    p = p_ref[0]
    sv = sv_ref[0]
    x = jax.nn.sigmoid(p[0:1, :])
    y = jax.nn.sigmoid(p[1:2, :])
    w = p[2:3, :]
    h = p[3:4, :]
    conf = jax.nn.sigmoid(p[4:5, :])
    cls = jax.nn.sigmoid(p[5:85, :])
    bx = (x + gx_ref[0]) * sv
    by = (y + gy_ref[0]) * sv
    bw = (jnp.exp(w) * aw_ref[0]) * sv
    bh = (jnp.exp(h) * ah_ref[0]) * sv
    x1 = bx - bw * 0.5
    y1 = by - bh * 0.5
    x2 = bx + bw * 0.5
    y2 = by + bh * 0.5
    boxes_ref[0] = jnp.concatenate([x1, y1, x2, y2], axis=0)
    s = conf * cls
    s = jnp.where(s > _SCORE_THRESH, s, 0.0) * valid_ref[0]
    s_ref[0] = s

    def bis_body(_, lohi):
        lo, hi = lohi
        mid = (lo + hi) * 0.5
        cnt = jnp.sum(jnp.where(s > mid, 1.0, 0.0))
        good = cnt >= float(_PRE_REAL)
        return jnp.where(good, mid, lo), jnp.where(good, hi, mid)

    lo, _ = jax.lax.fori_loop(
        0, _BISECT_ITERS, bis_body,
        (jnp.float32(_SCORE_THRESH), jnp.float32(1.0)))
    t_ref[0] = jnp.full((1, 128), lo, jnp.float32)


def _decode(P, AW, AH, GX, GY, SV, VALID, *, interpret=False):
    return pl.pallas_call(
        _decode_body,
        grid=(_BS,),
        in_specs=[
            pl.BlockSpec((1, 85, _NP), lambda b: (b, 0, 0)),
            pl.BlockSpec((1, 1, _NP), lambda b: (b, 0, 0)),
            pl.BlockSpec((1, 1, _NP), lambda b: (b, 0, 0)),
            pl.BlockSpec((1, 1, _NP), lambda b: (b, 0, 0)),
            pl.BlockSpec((1, 1, _NP), lambda b: (b, 0, 0)),
            pl.BlockSpec((1, 1, _NP), lambda b: (0, 0, 0)),
            pl.BlockSpec((1, 1, _NP), lambda b: (0, 0, 0)),
        ],
        out_specs=[
            pl.BlockSpec((1, 4, _NP), lambda b: (b, 0, 0)),
            pl.BlockSpec((1, 80, _NP), lambda b: (b, 0, 0)),
            pl.BlockSpec((1, 1, 128), lambda b: (b, 0, 0)),
        ],
        out_shape=[
            jax.ShapeDtypeStruct((_BS, 4, _NP), jnp.float32),
            jax.ShapeDtypeStruct((_BS, 80, _NP), jnp.float32),
            jax.ShapeDtypeStruct((_BS, 1, 128), jnp.float32),
        ],
        interpret=interpret,
    )(P, AW, AH, GX, GY, SV, VALID)


def _nms_body(btT_ref, btC_ref, tsr_ref, tsc_ref, tlr_ref, tlc_ref, out_ref):
    f32 = jnp.float32
    btT = btT_ref[0]
    btC = btC_ref[0]
    tsr = tsr_ref[0]
    tsc = tsc_ref[0]
    tlr = tlr_ref[0]
    tlc = tlc_ref[0]

    offc = tlc * 4096.0
    offr = tlr * 4096.0
    x1c = btT[:, 0:1] + offc
    y1c = btT[:, 1:2] + offc
    x2c = btT[:, 2:3] + offc
    y2c = btT[:, 3:4] + offc
    x1r = btC[0:1, :] + offr
    y1r = btC[1:2, :] + offr
    x2r = btC[2:3, :] + offr
    y2r = btC[3:4, :] + offr

    area_c = jnp.maximum(x2c - x1c, 0.0) * jnp.maximum(y2c - y1c, 0.0)
    area_r = jnp.maximum(x2r - x1r, 0.0) * jnp.maximum(y2r - y1r, 0.0)
    ltx = jnp.maximum(x1c, x1r)
    lty = jnp.maximum(y1c, y1r)
    rbx = jnp.minimum(x2c, x2r)
    rby = jnp.minimum(y2c, y2r)
    iw = jnp.maximum(rbx - ltx, 0.0)
    ih = jnp.maximum(rby - lty, 0.0)
    inter = iw * ih
    iou = inter / (area_c + area_r - inter + 1e-9)
    iou_gt = iou > _NMS_THRESH

    ia = jax.lax.broadcasted_iota(jnp.int32, (_PRE, _PRE), 0)
    ib = jax.lax.broadcasted_iota(jnp.int32, (_PRE, _PRE), 1)
    M2 = jnp.where(iou_gt & (ib < ia), 1.0, 0.0).astype(f32)
    Mup = jnp.where(iou_gt & (ia < ib), 1.0, 0.0).astype(f32)

    init_c = jnp.where(tsc > _SCORE_THRESH, 1.0, 0.0).astype(f32)
    init_r = jnp.where(tsr > _SCORE_THRESH, 1.0, 0.0).astype(f32)

    def cond(carry):
        _, _, it, changed = carry
        return changed & (it < _PRE)

    def body(carry):
        kc, kr, it, _ = carry
        sup_c = jax.lax.dot(M2, kc, preferred_element_type=f32)
        sup_r = jax.lax.dot(kr, Mup, preferred_element_type=f32)
        kc2 = jnp.where(sup_c > 0.5, 0.0, init_c)
        kr2 = jnp.where(sup_r > 0.5, 0.0, init_r)
        changed = jnp.any(kc2 != kc)
        return kc2, kr2, it + 1, changed

    kc, kr, _, _ = jax.lax.while_loop(
        cond, body, (init_c, init_r, jnp.int32(0), jnp.bool_(True)))

    fs0_c = kc * tsc
    fs0_r = kr * tsr

    beats = (fs0_r > fs0_c) | ((fs0_r == fs0_c) & (ib < ia))
    rank_c = jnp.sum(jnp.where(beats, 1.0, 0.0), axis=1, keepdims=True)

    pcols = jax.lax.broadcasted_iota(jnp.int32, (_PRE, 256), 1).astype(f32)
    PT = jnp.where(rank_c == pcols, 1.0, 0.0).astype(f32)

    z = jnp.zeros((1, _PRE), f32)
    V = jnp.concatenate(
        [btC[0:1, :], btC[1:2, :], btC[2:3, :], btC[3:4, :],
         tlr, fs0_r, z, z], axis=0)
    out_ref[0] = jax.lax.dot(V, PT, preferred_element_type=f32)


def _nms(tbT, tbC, tsr, tsc, tlr, tlc, *, interpret=False):
    return pl.pallas_call(
        _nms_body,
        grid=(_BS,),
        in_specs=[
            pl.BlockSpec((1, _PRE, 4), lambda b: (b, 0, 0)),
            pl.BlockSpec((1, 4, _PRE), lambda b: (b, 0, 0)),
            pl.BlockSpec((1, 1, _PRE), lambda b: (b, 0, 0)),
            pl.BlockSpec((1, _PRE, 1), lambda b: (b, 0, 0)),
            pl.BlockSpec((1, 1, _PRE), lambda b: (b, 0, 0)),
            pl.BlockSpec((1, _PRE, 1), lambda b: (b, 0, 0)),
        ],
        out_specs=pl.BlockSpec((1, 8, 256), lambda b: (b, 0, 0)),
        out_shape=jax.ShapeDtypeStruct((_BS, 8, 256), jnp.float32),
        interpret=interpret,
    )(tbT, tbC, tsr, tsc, tlr, tlc)


def _compact_body(s_hbm, t_hbm, outS_hbm, outI_hbm, cnt_hbm,
                  rowbuf, tbuf, scorebuf, idxbuf, ncbuf, sharedc, allc):
    i32 = jnp.int32
    core = lax.axis_index("c")
    sub = lax.axis_index("s")
    img = core * 2 + sub // 8
    g = lax.rem(sub, 8)
    half = (sub // 8) * 8
    wid = core * 16 + sub
    iota16 = lax.iota(i32, 16)

    pltpu.sync_copy(t_hbm.at[pl.ds(img * 128, 128)], tbuf)
    tv = tbuf[pl.ds(0, 16)]

    def row_body(r, cur):
        c_cls = g * 10 + r
        pltpu.sync_copy(s_hbm.at[pl.ds((img * 80 + c_cls) * _NP, _NP)],
                        rowbuf)

        def vec_body(i, cur):
            v = rowbuf[pl.ds(i * 16, 16)]
            m = v > tv
            cur_c = jnp.minimum(cur, _LOCAL)
            key = jnp.where(m, iota16, iota16 + 16)
            _, vs = plsc.sort_key_val(key, v)
            flat = (i * 16 + iota16) * 80 + c_cls
            _, fs = plsc.sort_key_val(key, flat)
            cnt = plsc.all_reduce_population_count(m)
            wm = iota16 < cnt
            pos = cur_c + iota16
            plsc.store_scatter(scorebuf, [pos], vs, mask=wm)
            plsc.store_scatter(idxbuf, [pos], fs, mask=wm)
            return cur + jnp.max(cnt)

        return lax.fori_loop(0, _NP // 16, vec_body, cur)

    cur = lax.fori_loop(0, 10, row_body, jnp.int32(0))
    cur = jnp.minimum(cur, _LOCAL)

    cur8 = (cur + 7) & ~7
    padm = iota16 < (cur8 - cur)
    plsc.store_scatter(scorebuf, [cur + iota16],
                       jnp.zeros((16,), jnp.float32), mask=padm)
    plsc.store_scatter(idxbuf, [cur + iota16],
                       jnp.zeros((16,), i32) + (1 << 22), mask=padm)

    ncbuf[...] = lax.broadcast_in_dim(cur8, (16,), ())
    pltpu.sync_copy(ncbuf, cnt_hbm.at[pl.ds((img * 8 + g) * 16, 16)])
    pltpu.sync_copy(ncbuf, sharedc.at[wid])
    plsc.subcore_barrier()
    pltpu.sync_copy(sharedc, allc)
    base = jnp.zeros((16,), i32)
    grp0 = core * 16 + half
    me = core * 16 + sub
    for j in range(32):
        pred = (j >= grp0) & (j < me)
        predv = lax.broadcast_in_dim(pred, (16,), ())
        base = base + jnp.where(predv, allc[j], 0)
    base_s = pl.multiple_of(jnp.max(base), 8)
    gbase = img * _REG + base_s

    nch = (cur8 + 127) // 128
    for t in range(8):
        nch_t = jnp.where(g == t, nch, 0)

        def wk(k, _):
            pltpu.sync_copy(scorebuf.at[pl.ds(k * 128, 128)],
                            outS_hbm.at[pl.ds(gbase + k * 128, 128)])
            pltpu.sync_copy(idxbuf.at[pl.ds(k * 128, 128)],
                            outI_hbm.at[pl.ds(gbase + k * 128, 128)])
            return 0

        lax.fori_loop(0, nch_t, wk, 0)
        plsc.subcore_barrier()


def _compact(S, T):
    mesh = plsc.VectorSubcoreMesh(core_axis_name="c", subcore_axis_name="s")
    f = pl.kernel(
        _compact_body,
        out_type=[
            jax.ShapeDtypeStruct((_BS * _REG,), jnp.float32),
            jax.ShapeDtypeStruct((_BS * _REG,), jnp.int32),
            jax.ShapeDtypeStruct((_BS * 8 * 16,), jnp.int32),
        ],
        mesh=mesh,
        compiler_params=pltpu.CompilerParams(needs_layout_passes=False),
        scratch_types=[
            pltpu.VMEM((_NP,), jnp.float32),
            pltpu.VMEM((128,), jnp.float32),
            pltpu.VMEM((_LOCAL + 16,), jnp.float32),
            pltpu.VMEM((_LOCAL + 16,), jnp.int32),
            pltpu.VMEM((16,), jnp.int32),
            pltpu.VMEM_SHARED((32, 16), jnp.int32),
            pltpu.VMEM((32, 16), jnp.int32),
        ],
    )
    return f(S, T)


def _rank_body(sr_ref, sc_ref, ir_ref, ic_ref, cnt_ref, out_ref):
    f32 = jnp.float32
    total = jnp.sum(cnt_ref[0])
    slot_r = jax.lax.broadcasted_iota(jnp.int32, (1, _CAP), 1)
    slot_c = jax.lax.broadcasted_iota(jnp.int32, (_CAP, 1), 0)
    vr = slot_r < total
    vc = slot_c < total
    sr = jnp.where(vr, sr_ref[0], 0.0)
    sc = jnp.where(vc, sc_ref[0], 0.0)
    ir = jnp.where(vr, ir_ref[0], slot_r.astype(f32))
    ic = jnp.where(vc, ic_ref[0], slot_c.astype(f32))
    beats = (sr > sc) | ((sr == sc) & (ir < ic))
    rank_c = jnp.sum(jnp.where(beats, 1.0, 0.0), axis=1, keepdims=True)
    pcols = jax.lax.broadcasted_iota(jnp.int32, (_CAP, _PRE), 1).astype(f32)
    PT = jnp.where(rank_c == pcols, 1.0, 0.0).astype(f32)
    srt_s = jax.lax.dot(sr, PT, preferred_element_type=f32)
    srt_i = jax.lax.dot(ir, PT, preferred_element_type=f32)
    out_ref[0] = jnp.concatenate([srt_s, srt_i], axis=0)


def _rank(sr, sc, ir, ic, cnt):
    return pl.pallas_call(
        _rank_body,
        grid=(_BS,),
        in_specs=[
            pl.BlockSpec((1, 1, _CAP), lambda b: (b, 0, 0)),
            pl.BlockSpec((1, _CAP, 1), lambda b: (b, 0, 0)),
            pl.BlockSpec((1, 1, _CAP), lambda b: (b, 0, 0)),
            pl.BlockSpec((1, _CAP, 1), lambda b: (b, 0, 0)),
            pl.BlockSpec((1, 1, 8), lambda b: (b, 0, 0)),
        ],
        out_specs=pl.BlockSpec((1, 2, _PRE), lambda b: (b, 0, 0)),
        out_shape=jax.ShapeDtypeStruct((_BS, 2, _PRE), jnp.float32),
    )(sr, sc, ir, ic, cnt)


def _prep_inputs(args):
    Ps, AWs, AHs, GXs, GYs = [], [], [], [], []
    for i, s in enumerate(_SIZES):
        inp, aw, ah, gx, gy = args[5 * i:5 * i + 5]
        p = inp.reshape(_BS, _NA, _NUM_CLASSES + 5, s, s)
        p = p.transpose(0, 2, 1, 3, 4).reshape(_BS, _NUM_CLASSES + 5, -1)
        Ps.append(p)
        AWs.append(aw.reshape(_BS, -1))
        AHs.append(ah.reshape(_BS, -1))
        GXs.append(gx.reshape(_BS, -1))
        GYs.append(gy.reshape(_BS, -1))
    P = jnp.concatenate(Ps, axis=2)
    pad = _NP - _NV
    P = jnp.pad(P, ((0, 0), (0, 0), (0, pad)))
    AW = jnp.pad(jnp.concatenate(AWs, axis=1), ((0, 0), (0, pad)))[:, None, :]
    AH = jnp.pad(jnp.concatenate(AHs, axis=1), ((0, 0), (0, pad)))[:, None, :]
    GX = jnp.pad(jnp.concatenate(GXs, axis=1), ((0, 0), (0, pad)))[:, None, :]
    GY = jnp.pad(jnp.concatenate(GYs, axis=1), ((0, 0), (0, pad)))[:, None, :]
    sv = np.concatenate([
        np.full(_NA * s * s, st, np.float32)
        for s, st in zip(_SIZES, _STRIDES)] + [np.ones(pad, np.float32)])
    valid = np.concatenate(
        [np.ones(_NV, np.float32), np.zeros(pad, np.float32)])
    SV = jnp.asarray(sv)[None, None, :]
    VALID = jnp.asarray(valid)[None, None, :]
    return P, AW, AH, GX, GY, SV, VALID


def _pipeline(args, interpret=False):
    P, AW, AH, GX, GY, SV, VALID = _prep_inputs(args)
    boxes, S, T = _decode(P, AW, AH, GX, GY, SV, VALID, interpret=interpret)

    outS, outI, cnts = _compact(S.reshape(-1), T.reshape(-1))
    S2 = outS.reshape(_BS, _REG)[:, :_CAP]
    I2 = outI.reshape(_BS, _REG)[:, :_CAP].astype(jnp.float32)
    C2 = cnts.reshape(_BS, 8, 16)[:, :, 0][:, None, :]
    out2 = _rank(S2[:, None, :], S2[:, :, None],
                 I2[:, None, :], I2[:, :, None], C2)
    ts = out2[:, 0, :_PRE_REAL]
    ti = out2[:, 1, :_PRE_REAL].astype(jnp.int32)
    n = ti // _NUM_CLASSES
    c = ti % _NUM_CLASSES
    boxesT = boxes.transpose(0, 2, 1)
    tb = jnp.take_along_axis(boxesT, n[..., None], axis=1)
    tl = c.astype(jnp.float32)

    padk = _PRE - _PRE_REAL
    tbT = jnp.pad(tb, ((0, 0), (0, padk), (0, 0)))
    ts_p = jnp.pad(ts, ((0, 0), (0, padk)))
    tl_p = jnp.pad(tl, ((0, 0), (0, padk)))
    tbC = tbT.transpose(0, 2, 1)
    tsr = ts_p[:, None, :]
    tsc = ts_p[:, :, None]
    tlr = tl_p[:, None, :]
    tlc = tl_p[:, :, None]

    out = _nms(tbT, tbC, tsr, tsc, tlr, tlc, interpret=interpret)
    return out.transpose(0, 2, 1)[:, :_TOPK, :6]


def kernel(input_l0, anchor_w_l0, anchor_h_l0, grid_x_l0, grid_y_l0,
           input_l1, anchor_w_l1, anchor_h_l1, grid_x_l1, grid_y_l1,
           input_l2, anchor_w_l2, anchor_h_l2, grid_x_l2, grid_y_l2):
    args = (input_l0, anchor_w_l0, anchor_h_l0, grid_x_l0, grid_y_l0,
            input_l1, anchor_w_l1, anchor_h_l1, grid_x_l1, grid_y_l1,
            input_l2, anchor_w_l2, anchor_h_l2, grid_x_l2, grid_y_l2)
    return _pipeline(args)


# speedup vs baseline: 1.0012x; 1.0012x over previous
import jax
import jax.numpy as jnp
import numpy as np
from jax import lax
from jax.experimental import pallas as pl
from jax.experimental.pallas import tpu as pltpu
from jax.experimental.pallas import tpu_sc as plsc

_NUM_CLASSES = 80
_STRIDES = (32.0, 16.0, 8.0)
_SIZES = (13, 26, 52)
_NA = 3
_BS = 4
_SCORE_THRESH = 0.01
_NMS_THRESH = 0.45
_PRE = 1024
_PRE_REAL = 1000
_TOPK = 200
_NV = sum(_NA * s * s for s in _SIZES)
_NP = 10752
_CAP = 2048
_LOCAL = 2048
_REG = _CAP + 2048
_BISECT_ITERS = 14


def _decode_body(p_ref, aw_ref, ah_ref, gx_ref, gy_ref, sv_ref, valid_ref,
                 boxes_ref, s_ref, t_ref):
    p = p_ref[0]
    sv = sv_ref[0]
    x = jax.nn.sigmoid(p[0:1, :])
    y = jax.nn.sigmoid(p[1:2, :])
    w = p[2:3, :]
    h = p[3:4, :]
    conf = jax.nn.sigmoid(p[4:5, :])
    cls = jax.nn.sigmoid(p[5:85, :])
    bx = (x + gx_ref[0]) * sv
    by = (y + gy_ref[0]) * sv
    bw = (jnp.exp(w) * aw_ref[0]) * sv
    bh = (jnp.exp(h) * ah_ref[0]) * sv
    x1 = bx - bw * 0.5
    y1 = by - bh * 0.5
    x2 = bx + bw * 0.5
    y2 = by + bh * 0.5
    boxes_ref[0] = jnp.concatenate([x1, y1, x2, y2], axis=0)
    s = conf * cls
    s = jnp.where(s > _SCORE_THRESH, s, 0.0) * valid_ref[0]
    s_ref[0] = s

    def bis_body(_, lohi):
        lo, hi = lohi
        mid = (lo + hi) * 0.5
        cnt = jnp.sum(jnp.where(s > mid, 1.0, 0.0))
        good = cnt >= float(_PRE_REAL)
        return jnp.where(good, mid, lo), jnp.where(good, hi, mid)

    lo, _ = jax.lax.fori_loop(
        0, _BISECT_ITERS, bis_body,
        (jnp.float32(_SCORE_THRESH), jnp.float32(1.0)))
    t_ref[0] = jnp.full((1, 128), lo, jnp.float32)


def _decode(P, AW, AH, GX, GY, SV, VALID, *, interpret=False):
    return pl.pallas_call(
        _decode_body,
        grid=(_BS,),
        in_specs=[
            pl.BlockSpec((1, 85, _NP), lambda b: (b, 0, 0)),
            pl.BlockSpec((1, 1, _NP), lambda b: (b, 0, 0)),
            pl.BlockSpec((1, 1, _NP), lambda b: (b, 0, 0)),
            pl.BlockSpec((1, 1, _NP), lambda b: (b, 0, 0)),
            pl.BlockSpec((1, 1, _NP), lambda b: (b, 0, 0)),
            pl.BlockSpec((1, 1, _NP), lambda b: (0, 0, 0)),
            pl.BlockSpec((1, 1, _NP), lambda b: (0, 0, 0)),
        ],
        out_specs=[
            pl.BlockSpec((1, 4, _NP), lambda b: (b, 0, 0)),
            pl.BlockSpec((1, 80, _NP), lambda b: (b, 0, 0)),
            pl.BlockSpec((1, 1, 128), lambda b: (b, 0, 0)),
        ],
        out_shape=[
            jax.ShapeDtypeStruct((_BS, 4, _NP), jnp.float32),
            jax.ShapeDtypeStruct((_BS, 80, _NP), jnp.float32),
            jax.ShapeDtypeStruct((_BS, 1, 128), jnp.float32),
        ],
        interpret=interpret,
    )(P, AW, AH, GX, GY, SV, VALID)


def _nms_body(btT_ref, btC_ref, tsr_ref, tsc_ref, tlr_ref, tlc_ref, out_ref):
    f32 = jnp.float32
    btT = btT_ref[0]
    btC = btC_ref[0]
    tsr = tsr_ref[0]
    tsc = tsc_ref[0]
    tlr = tlr_ref[0]
    tlc = tlc_ref[0]

    offc = tlc * 4096.0
    offr = tlr * 4096.0
    x1c = btT[:, 0:1] + offc
    y1c = btT[:, 1:2] + offc
    x2c = btT[:, 2:3] + offc
    y2c = btT[:, 3:4] + offc
    x1r = btC[0:1, :] + offr
    y1r = btC[1:2, :] + offr
    x2r = btC[2:3, :] + offr
    y2r = btC[3:4, :] + offr

    area_c = jnp.maximum(x2c - x1c, 0.0) * jnp.maximum(y2c - y1c, 0.0)
    area_r = jnp.maximum(x2r - x1r, 0.0) * jnp.maximum(y2r - y1r, 0.0)
    ltx = jnp.maximum(x1c, x1r)
    lty = jnp.maximum(y1c, y1r)
    rbx = jnp.minimum(x2c, x2r)
    rby = jnp.minimum(y2c, y2r)
    iw = jnp.maximum(rbx - ltx, 0.0)
    ih = jnp.maximum(rby - lty, 0.0)
    inter = iw * ih
    iou = inter / (area_c + area_r - inter + 1e-9)
    iou_gt = iou > _NMS_THRESH

    ia = jax.lax.broadcasted_iota(jnp.int32, (_PRE, _PRE), 0)
    ib = jax.lax.broadcasted_iota(jnp.int32, (_PRE, _PRE), 1)
    M2 = jnp.where(iou_gt & (ib < ia), 1.0, 0.0).astype(f32)
    Mup = jnp.where(iou_gt & (ia < ib), 1.0, 0.0).astype(f32)

    init_c = jnp.where(tsc > _SCORE_THRESH, 1.0, 0.0).astype(f32)
    init_r = jnp.where(tsr > _SCORE_THRESH, 1.0, 0.0).astype(f32)

    def cond(carry):
        _, _, it, changed = carry
        return changed & (it < _PRE)

    def body(carry):
        kc, kr, it, _ = carry
        sup_c = jax.lax.dot(M2, kc, preferred_element_type=f32)
        sup_r = jax.lax.dot(kr, Mup, preferred_element_type=f32)
        kc2 = jnp.where(sup_c > 0.5, 0.0, init_c)
        kr2 = jnp.where(sup_r > 0.5, 0.0, init_r)
        changed = jnp.any(kc2 != kc)
        return kc2, kr2, it + 1, changed

    kc, kr, _, _ = jax.lax.while_loop(
        cond, body, (init_c, init_r, jnp.int32(0), jnp.bool_(True)))

    fs0_c = kc * tsc
    fs0_r = kr * tsr

    beats = (fs0_r > fs0_c) | ((fs0_r == fs0_c) & (ib < ia))
    rank_c = jnp.sum(jnp.where(beats, 1.0, 0.0), axis=1, keepdims=True)

    pcols = jax.lax.broadcasted_iota(jnp.int32, (_PRE, 256), 1).astype(f32)
    PT = jnp.where(rank_c == pcols, 1.0, 0.0).astype(f32)

    z = jnp.zeros((1, _PRE), f32)
    V = jnp.concatenate(
        [btC[0:1, :], btC[1:2, :], btC[2:3, :], btC[3:4, :],
         tlr, fs0_r, z, z], axis=0)
    out_ref[0] = jax.lax.dot(V, PT, preferred_element_type=f32)


def _nms(tbT, tbC, tsr, tsc, tlr, tlc, *, interpret=False):
    return pl.pallas_call(
        _nms_body,
        grid=(_BS,),
        in_specs=[
            pl.BlockSpec((1, _PRE, 4), lambda b: (b, 0, 0)),
            pl.BlockSpec((1, 4, _PRE), lambda b: (b, 0, 0)),
            pl.BlockSpec((1, 1, _PRE), lambda b: (b, 0, 0)),
            pl.BlockSpec((1, _PRE, 1), lambda b: (b, 0, 0)),
            pl.BlockSpec((1, 1, _PRE), lambda b: (b, 0, 0)),
            pl.BlockSpec((1, _PRE, 1), lambda b: (b, 0, 0)),
        ],
        out_specs=pl.BlockSpec((1, 8, 256), lambda b: (b, 0, 0)),
        out_shape=jax.ShapeDtypeStruct((_BS, 8, 256), jnp.float32),
        interpret=interpret,
    )(tbT, tbC, tsr, tsc, tlr, tlc)


def _compact_body(s_hbm, t_hbm, outS_hbm, outI_hbm, cnt_hbm,
                  rowbuf, tbuf, scorebuf, idxbuf, ncbuf, sharedc, allc):
    i32 = jnp.int32
    core = lax.axis_index("c")
    sub = lax.axis_index("s")
    img = core * 2 + sub // 8
    g = lax.rem(sub, 8)
    half = (sub // 8) * 8
    wid = core * 16 + sub
    iota16 = lax.iota(i32, 16)

    pltpu.sync_copy(t_hbm.at[pl.ds(img * 128, 128)], tbuf)
    tv = tbuf[pl.ds(0, 16)]

    def row_body(r, cur):
        c_cls = g * 10 + r
        pltpu.sync_copy(s_hbm.at[pl.ds((img * 80 + c_cls) * _NP, _NP)],
                        rowbuf)

        def vec_body(i, cur):
            v = rowbuf[pl.ds(i * 16, 16)]
            m = v > tv
            cur_c = jnp.minimum(cur, _LOCAL)
            key = jnp.where(m, iota16, iota16 + 16)
            _, vs = plsc.sort_key_val(key, v)
            flat = (i * 16 + iota16) * 80 + c_cls
            _, fs = plsc.sort_key_val(key, flat)
            cnt = plsc.all_reduce_population_count(m)
            wm = iota16 < cnt
            pos = cur_c + iota16
            plsc.store_scatter(scorebuf, [pos], vs, mask=wm)
            plsc.store_scatter(idxbuf, [pos], fs, mask=wm)
            return cur + jnp.max(cnt)

        return lax.fori_loop(0, _NP // 16, vec_body, cur)

    cur = lax.fori_loop(0, 10, row_body, jnp.int32(0))
    cur = jnp.minimum(cur, _LOCAL)

    cur8 = (cur + 7) & ~7
    padm = iota16 < (cur8 - cur)
    plsc.store_scatter(scorebuf, [cur + iota16],
                       jnp.zeros((16,), jnp.float32), mask=padm)
    plsc.store_scatter(idxbuf, [cur + iota16],
                       jnp.zeros((16,), i32) + (1 << 22), mask=padm)

    ncbuf[...] = lax.broadcast_in_dim(cur8, (16,), ())
    pltpu.sync_copy(ncbuf, cnt_hbm.at[pl.ds((img * 8 + g) * 16, 16)])
    pltpu.sync_copy(ncbuf, sharedc.at[wid])
    plsc.subcore_barrier()
    pltpu.sync_copy(sharedc, allc)
    base = jnp.zeros((16,), i32)
    grp0 = core * 16 + half
    me = core * 16 + sub
    for j in range(32):
        pred = (j >= grp0) & (j < me)
        predv = lax.broadcast_in_dim(pred, (16,), ())
        base = base + jnp.where(predv, allc[j], 0)
    base_s = pl.multiple_of(jnp.max(base), 8)
    gbase = img * _REG + base_s

    nch = (cur8 + 127) // 128
    for t in range(8):
        nch_t = jnp.where(g == t, nch, 0)

        def wk(k, _):
            pltpu.sync_copy(scorebuf.at[pl.ds(k * 128, 128)],
                            outS_hbm.at[pl.ds(gbase + k * 128, 128)])
            pltpu.sync_copy(idxbuf.at[pl.ds(k * 128, 128)],
                            outI_hbm.at[pl.ds(gbase + k * 128, 128)])
            return 0

        lax.fori_loop(0, nch_t, wk, 0)
        plsc.subcore_barrier()


def _compact(S, T):
    mesh = plsc.VectorSubcoreMesh(core_axis_name="c", subcore_axis_name="s")
    f = pl.kernel(
        _compact_body,
        out_type=[
            jax.ShapeDtypeStruct((_BS * _REG,), jnp.float32),
            jax.ShapeDtypeStruct((_BS * _REG,), jnp.int32),
            jax.ShapeDtypeStruct((_BS * 8 * 16,), jnp.int32),
        ],
        mesh=mesh,
        compiler_params=pltpu.CompilerParams(needs_layout_passes=False),
        scratch_types=[
            pltpu.VMEM((_NP,), jnp.float32),
            pltpu.VMEM((128,), jnp.float32),
            pltpu.VMEM((_LOCAL + 16,), jnp.float32),
            pltpu.VMEM((_LOCAL + 16,), jnp.int32),
            pltpu.VMEM((16,), jnp.int32),
            pltpu.VMEM_SHARED((32, 16), jnp.int32),
            pltpu.VMEM((32, 16), jnp.int32),
        ],
    )
    return f(S, T)


def _rank_body(sr_ref, sc_ref, ir_ref, ic_ref, cnt_ref, out_ref):
    f32 = jnp.float32
    total = jnp.sum(cnt_ref[0])
    slot_r = jax.lax.broadcasted_iota(jnp.int32, (1, _CAP), 1)
    slot_c = jax.lax.broadcasted_iota(jnp.int32, (_CAP, 1), 0)
    vr = slot_r < total
    vc = slot_c < total
    sr = jnp.where(vr, sr_ref[0], 0.0)
    sc = jnp.where(vc, sc_ref[0], 0.0)
    ir = jnp.where(vr, ir_ref[0], slot_r.astype(f32))
    ic = jnp.where(vc, ic_ref[0], slot_c.astype(f32))
    beats = (sr > sc) | ((sr == sc) & (ir < ic))
    rank_c = jnp.sum(jnp.where(beats, 1.0, 0.0), axis=1, keepdims=True)
    pcols = jax.lax.broadcasted_iota(jnp.int32, (_CAP, _PRE), 1).astype(f32)
    PT = jnp.where(rank_c == pcols, 1.0, 0.0).astype(f32)
    srt_s = jax.lax.dot(sr, PT, preferred_element_type=f32)
    srt_i = jax.lax.dot(ir, PT, preferred_element_type=f32)
    out_ref[0] = jnp.concatenate([srt_s, srt_i], axis=0)


def _rank(sr, sc, ir, ic, cnt):
    return pl.pallas_call(
        _rank_body,
        grid=(_BS,),
        in_specs=[
            pl.BlockSpec((1, 1, _CAP), lambda b: (b, 0, 0)),
            pl.BlockSpec((1, _CAP, 1), lambda b: (b, 0, 0)),
            pl.BlockSpec((1, 1, _CAP), lambda b: (b, 0, 0)),
            pl.BlockSpec((1, _CAP, 1), lambda b: (b, 0, 0)),
            pl.BlockSpec((1, 1, 8), lambda b: (b, 0, 0)),
        ],
        out_specs=pl.BlockSpec((1, 2, _PRE), lambda b: (b, 0, 0)),
        out_shape=jax.ShapeDtypeStruct((_BS, 2, _PRE), jnp.float32),
    )(sr, sc, ir, ic, cnt)


def _prep_inputs(args):
    Ps, AWs, AHs, GXs, GYs = [], [], [], [], []
    for i, s in enumerate(_SIZES):
        inp, aw, ah, gx, gy = args[5 * i:5 * i + 5]
        p = inp.reshape(_BS, _NA, _NUM_CLASSES + 5, s, s)
        p = p.transpose(0, 2, 1, 3, 4).reshape(_BS, _NUM_CLASSES + 5, -1)
        Ps.append(p)
        AWs.append(aw.reshape(_BS, -1))
        AHs.append(ah.reshape(_BS, -1))
        GXs.append(gx.reshape(_BS, -1))
        GYs.append(gy.reshape(_BS, -1))
    P = jnp.concatenate(Ps, axis=2)
    pad = _NP - _NV
    P = jnp.pad(P, ((0, 0), (0, 0), (0, pad)))
    AW = jnp.pad(jnp.concatenate(AWs, axis=1), ((0, 0), (0, pad)))[:, None, :]
    AH = jnp.pad(jnp.concatenate(AHs, axis=1), ((0, 0), (0, pad)))[:, None, :]
    GX = jnp.pad(jnp.concatenate(GXs, axis=1), ((0, 0), (0, pad)))[:, None, :]
    GY = jnp.pad(jnp.concatenate(GYs, axis=1), ((0, 0), (0, pad)))[:, None, :]
    sv = np.concatenate([
        np.full(_NA * s * s, st, np.float32)
        for s, st in zip(_SIZES, _STRIDES)] + [np.ones(pad, np.float32)])
    valid = np.concatenate(
        [np.ones(_NV, np.float32), np.zeros(pad, np.float32)])
    SV = jnp.asarray(sv)[None, None, :]
    VALID = jnp.asarray(valid)[None, None, :]
    return P, AW, AH, GX, GY, SV, VALID


def _pipeline(args, interpret=False):
    P, AW, AH, GX, GY, SV, VALID = _prep_inputs(args)
    boxes, S, T = _decode(P, AW, AH, GX, GY, SV, VALID, interpret=interpret)

    outS, outI, cnts = _compact(S.reshape(-1), T.reshape(-1))
    S2 = outS.reshape(_BS, _REG)[:, :_CAP]
    I2 = outI.reshape(_BS, _REG)[:, :_CAP].astype(jnp.float32)
    C2 = cnts.reshape(_BS, 8, 16)[:, :, 0][:, None, :]
    out2 = _rank(S2[:, None, :], S2[:, :, None],
                 I2[:, None, :], I2[:, :, None], C2)
    ts = out2[:, 0, :_PRE_REAL]
    ti = out2[:, 1, :_PRE_REAL].astype(jnp.int32)
    n = ti // _NUM_CLASSES
    c = ti % _NUM_CLASSES
    boxesT = boxes.transpose(0, 2, 1)
    tb = jnp.take_along_axis(boxesT, n[..., None], axis=1)
    tl = c.astype(jnp.float32)

    padk = _PRE - _PRE_REAL
    tbT = jnp.pad(tb, ((0, 0), (0, padk), (0, 0)))
    ts_p = jnp.pad(ts, ((0, 0), (0, padk)))
    tl_p = jnp.pad(tl, ((0, 0), (0, padk)))
    tbC = tbT.transpose(0, 2, 1)
    tsr = ts_p[:, None, :]
    tsc = ts_p[:, :, None]
    tlr = tl_p[:, None, :]
    tlc = tl_p[:, :, None]

    out = _nms(tbT, tbC, tsr, tsc, tlr, tlc, interpret=interpret)
    return out.transpose(0, 2, 1)[:, :_TOPK, :6]


def kernel(input_l0, anchor_w_l0, anchor_h_l0, grid_x_l0, grid_y_l0,
           input_l1, anchor_w_l1, anchor_h_l1, grid_x_l1, grid_y_l1,
           input_l2, anchor_w_l2, anchor_h_l2, grid_x_l2, grid_y_l2):
    args = (input_l0, anchor_w_l0, anchor_h_l0, grid_x_l0, grid_y_l0,
            input_l1, anchor_w_l1, anchor_h_l1, grid_x_l1, grid_y_l1,
            input_l2, anchor_w_l2, anchor_h_l2, grid_x_l2, grid_y_l2)
    return _pipeline(args)
